# initial kernel scaffold (unmeasured)
import jax
import jax.numpy as jnp
from jax import lax
from jax.experimental import pallas as pl
from jax.experimental.pallas import tpu as pltpu


def kernel(x, assign, W1, W2):
    T, D = x.shape
    E, _, F = W1.shape
    assign2 = assign.reshape(T, 1)

    def body(x_ref, a_ref, w1_ref, w2_ref, out_ref,
             xsend, xrecv, arecv, prsend, prrecv, send_sems, recv_sems):
        my_x = lax.axis_index("x")
        my_y = lax.axis_index("y")
        peer = (1 - my_x, my_y)

        xsend[...] = x_ref[...].astype(jnp.bfloat16)
        cx = pltpu.make_async_remote_copy(
            src_ref=xsend, dst_ref=xrecv,
            send_sem=send_sems.at[0], recv_sem=recv_sems.at[0],
            device_id=peer, device_id_type=pl.DeviceIdType.MESH,
        )
        cx.start()
        ca = pltpu.make_async_remote_copy(
            src_ref=a_ref, dst_ref=arecv,
            send_sem=send_sems.at[1], recv_sem=recv_sems.at[1],
            device_id=peer, device_id_type=pl.DeviceIdType.MESH,
        )
        ca.start()

        def moe_local_experts(xb, ab):
            acc = jnp.zeros((T, D), jnp.float32)
            for e in range(E):
                eid = my_x * E + e
                xm = jnp.where(ab == eid, xb, jnp.bfloat16(0.0))
                h = jnp.dot(xm, w1_ref[e].astype(jnp.bfloat16),
                            preferred_element_type=jnp.float32)
                h = jnp.maximum(h, 0.0).astype(jnp.bfloat16)
                acc = acc + jnp.dot(h, w2_ref[e].astype(jnp.bfloat16),
                                    preferred_element_type=jnp.float32)
            return acc

        out_ref[...] = moe_local_experts(xsend[...], a_ref[...])

        cx.wait()
        ca.wait()

        prsend[...] = moe_local_experts(
            xrecv[...], arecv[...]).astype(jnp.bfloat16)
        cp = pltpu.make_async_remote_copy(
            src_ref=prsend, dst_ref=prrecv,
            send_sem=send_sems.at[2], recv_sem=recv_sems.at[2],
            device_id=peer, device_id_type=pl.DeviceIdType.MESH,
        )
        cp.start()
        cp.wait()

        out_ref[...] = out_ref[...] + prrecv[...].astype(jnp.float32)

    out_shape = jax.ShapeDtypeStruct((T, D), jnp.float32)
    return pl.pallas_call(
        body,
        out_shape=out_shape,
        in_specs=[
            pl.BlockSpec(memory_space=pltpu.VMEM),
            pl.BlockSpec(memory_space=pltpu.VMEM),
            pl.BlockSpec(memory_space=pltpu.VMEM),
            pl.BlockSpec(memory_space=pltpu.VMEM),
        ],
        out_specs=pl.BlockSpec(memory_space=pltpu.VMEM),
        scratch_shapes=[
            pltpu.VMEM((T, D), jnp.bfloat16),
            pltpu.VMEM((T, D), jnp.bfloat16),
            pltpu.VMEM((T, 1), jnp.int32),
            pltpu.VMEM((T, D), jnp.bfloat16),
            pltpu.VMEM((T, D), jnp.bfloat16),
            pltpu.SemaphoreType.DMA((3,)),
            pltpu.SemaphoreType.DMA((3,)),
        ],
        compiler_params=pltpu.CompilerParams(collective_id=0),
    )(x, assign2, W1, W2)


# baseline (device time: 32315 ns/iter reference)
import jax
import jax.numpy as jnp
from jax import lax
from jax.experimental import pallas as pl
from jax.experimental.pallas import tpu as pltpu


def kernel(x, assign, W1, W2):
    T, D = x.shape
    E, _, F = W1.shape
    assign2 = assign.reshape(T, 1)

    def body(x_ref, a_ref, w1_ref, w2_ref, out_ref,
             xsend, xrecv, arecv, prsend, prrecv, send_sems, recv_sems):
        my_x = lax.axis_index("x")
        my_y = lax.axis_index("y")
        peer = (1 - my_x, my_y)

        xsend[...] = x_ref[...].astype(jnp.bfloat16)
        cx = pltpu.make_async_remote_copy(
            src_ref=xsend, dst_ref=xrecv,
            send_sem=send_sems.at[0], recv_sem=recv_sems.at[0],
            device_id=peer, device_id_type=pl.DeviceIdType.MESH,
        )
        cx.start()
        ca = pltpu.make_async_remote_copy(
            src_ref=a_ref, dst_ref=arecv,
            send_sem=send_sems.at[1], recv_sem=recv_sems.at[1],
            device_id=peer, device_id_type=pl.DeviceIdType.MESH,
        )
        ca.start()

        def moe_local_experts(xb, ab):
            acc = jnp.zeros((T, D), jnp.float32)
            for e in range(E):
                eid = my_x * E + e
                xm = jnp.where(ab == eid, xb, jnp.bfloat16(0.0))
                h = jnp.dot(xm, w1_ref[e].astype(jnp.bfloat16),
                            preferred_element_type=jnp.float32)
                h = jnp.maximum(h, 0.0).astype(jnp.bfloat16)
                acc = acc + jnp.dot(h, w2_ref[e].astype(jnp.bfloat16),
                                    preferred_element_type=jnp.float32)
            return acc

        out_ref[...] = moe_local_experts(xsend[...], a_ref[...])

        cx.wait()
        ca.wait()

        prsend[...] = moe_local_experts(
            xrecv[...], arecv[...]).astype(jnp.bfloat16)
        cp = pltpu.make_async_remote_copy(
            src_ref=prsend, dst_ref=prrecv,
            send_sem=send_sems.at[2], recv_sem=recv_sems.at[2],
            device_id=peer, device_id_type=pl.DeviceIdType.MESH,
        )
        cp.start()
        cp.wait()

        out_ref[...] = out_ref[...] + prrecv[...].astype(jnp.float32)

    out_shape = jax.ShapeDtypeStruct((T, D), jnp.float32)
    return pl.pallas_call(
        body,
        out_shape=out_shape,
        in_specs=[
            pl.BlockSpec(memory_space=pltpu.VMEM),
            pl.BlockSpec(memory_space=pltpu.VMEM),
            pl.BlockSpec(memory_space=pltpu.VMEM),
            pl.BlockSpec(memory_space=pltpu.VMEM),
        ],
        out_specs=pl.BlockSpec(memory_space=pltpu.VMEM),
        scratch_shapes=[
            pltpu.VMEM((T, D), jnp.bfloat16),
            pltpu.VMEM((T, D), jnp.bfloat16),
            pltpu.VMEM((T, 1), jnp.int32),
            pltpu.VMEM((T, D), jnp.bfloat16),
            pltpu.VMEM((T, D), jnp.bfloat16),
            pltpu.SemaphoreType.DMA((3,)),
            pltpu.SemaphoreType.DMA((3,)),
        ],
    )(x, assign2, W1, W2)


# device time: 29600 ns/iter; 1.0917x vs baseline; 1.0917x over previous
import jax
import jax.numpy as jnp
from jax import lax
from jax.experimental import pallas as pl
from jax.experimental.pallas import tpu as pltpu


def kernel(x, assign, W1, W2):
    T, D = x.shape
    E, _, F = W1.shape
    H = T // 2
    assign2 = assign.reshape(T, 1)

    def body(x_ref, a_ref, w1_ref, w2_ref, out_ref,
             xsend, asend, xrecv, arecv, prsend, prrecv, fsend, frecv,
             send_sems, recv_sems):
        my_x = lax.axis_index("x")
        my_y = lax.axis_index("y")
        xpeer = (1 - my_x, my_y)
        ypeer = (my_x, 1 - my_y)
        h0 = my_y * H

        xsend[...] = x_ref[pl.ds(h0, H), :].astype(jnp.bfloat16)
        asend[...] = a_ref[pl.ds(h0, H), :]
        cx = pltpu.make_async_remote_copy(
            src_ref=xsend, dst_ref=xrecv,
            send_sem=send_sems.at[0], recv_sem=recv_sems.at[0],
            device_id=xpeer, device_id_type=pl.DeviceIdType.MESH,
        )
        cx.start()
        ca = pltpu.make_async_remote_copy(
            src_ref=asend, dst_ref=arecv,
            send_sem=send_sems.at[1], recv_sem=recv_sems.at[1],
            device_id=xpeer, device_id_type=pl.DeviceIdType.MESH,
        )
        ca.start()

        def moe_local_experts(xb, ab):
            acc = jnp.zeros((H, D), jnp.float32)
            for e in range(E):
                eid = my_x * E + e
                xm = jnp.where(ab == eid, xb, jnp.bfloat16(0.0))
                h = jnp.dot(xm, w1_ref[e].astype(jnp.bfloat16),
                            preferred_element_type=jnp.float32)
                h = jnp.maximum(h, 0.0).astype(jnp.bfloat16)
                acc = acc + jnp.dot(h, w2_ref[e].astype(jnp.bfloat16),
                                    preferred_element_type=jnp.float32)
            return acc

        p_own = moe_local_experts(xsend[...], asend[...])

        cx.wait()
        ca.wait()

        prsend[...] = moe_local_experts(
            xrecv[...], arecv[...]).astype(jnp.bfloat16)
        cp = pltpu.make_async_remote_copy(
            src_ref=prsend, dst_ref=prrecv,
            send_sem=send_sems.at[2], recv_sem=recv_sems.at[2],
            device_id=xpeer, device_id_type=pl.DeviceIdType.MESH,
        )
        cp.start()
        cp.wait()

        fin = p_own + prrecv[...].astype(jnp.float32)
        fsend[...] = fin.astype(jnp.bfloat16)
        cf = pltpu.make_async_remote_copy(
            src_ref=fsend, dst_ref=frecv,
            send_sem=send_sems.at[3], recv_sem=recv_sems.at[3],
            device_id=ypeer, device_id_type=pl.DeviceIdType.MESH,
        )
        cf.start()
        out_ref[pl.ds(h0, H), :] = fin
        cf.wait()
        out_ref[pl.ds((1 - my_y) * H, H), :] = frecv[...].astype(jnp.float32)

    out_shape = jax.ShapeDtypeStruct((T, D), jnp.float32)
    return pl.pallas_call(
        body,
        out_shape=out_shape,
        in_specs=[
            pl.BlockSpec(memory_space=pltpu.VMEM),
            pl.BlockSpec(memory_space=pltpu.VMEM),
            pl.BlockSpec(memory_space=pltpu.VMEM),
            pl.BlockSpec(memory_space=pltpu.VMEM),
        ],
        out_specs=pl.BlockSpec(memory_space=pltpu.VMEM),
        scratch_shapes=[
            pltpu.VMEM((H, D), jnp.bfloat16),
            pltpu.VMEM((H, 1), jnp.int32),
            pltpu.VMEM((H, D), jnp.bfloat16),
            pltpu.VMEM((H, 1), jnp.int32),
            pltpu.VMEM((H, D), jnp.bfloat16),
            pltpu.VMEM((H, D), jnp.bfloat16),
            pltpu.VMEM((H, D), jnp.bfloat16),
            pltpu.VMEM((H, D), jnp.bfloat16),
            pltpu.SemaphoreType.DMA((4,)),
            pltpu.SemaphoreType.DMA((4,)),
        ],
    )(x, assign2, W1, W2)


# device time: 7619 ns/iter; 4.2414x vs baseline; 3.8850x over previous
import os

import jax
import jax.numpy as jnp
from jax import lax
from jax.experimental import pallas as pl
from jax.experimental.pallas import tpu as pltpu

try:
    _MODE = open(os.path.join(os.path.dirname(__file__),
                              "kernel_mode.txt")).read().strip()
except OSError:
    _MODE = "full"

_DO_BAR = _MODE != "compute"
_DO_P1 = _MODE in ("comm1", "comm2", "comm3", "full")
_DO_P3 = _MODE in ("comm2", "comm3", "full")
_DO_P4 = _MODE in ("comm3", "full")
_DO_COMPUTE = _MODE in ("compute", "full")


def kernel(x, assign, W1, W2):
    if _MODE in ("nop", "nopw"):
        T, D = x.shape

        def zbody(*refs):
            refs[-1][...] = jnp.zeros((T, D), jnp.float32)

        ops = (x, assign.reshape(T, 1)) + (() if _MODE == "nop" else (W1, W2))
        return pl.pallas_call(
            zbody,
            out_shape=jax.ShapeDtypeStruct((T, D), jnp.float32),
            in_specs=[pl.BlockSpec(memory_space=pltpu.VMEM)] * len(ops),
            out_specs=pl.BlockSpec(memory_space=pltpu.VMEM),
        )(*ops)

    T, D = x.shape
    E, _, F = W1.shape
    H = T // 2
    assign2 = assign.reshape(T, 1)

    def body(x_ref, a_ref, w1_ref, w2_ref, out_ref,
             xsend, asend, xrecv, arecv, prsend, prrecv, fsend, frecv,
             send_sems, recv_sems):
        my_x = lax.axis_index("x")
        my_y = lax.axis_index("y")
        xpeer = (1 - my_x, my_y)
        ypeer = (my_x, 1 - my_y)
        h0 = my_y * H

        if _DO_BAR:
            barrier_sem = pltpu.get_barrier_semaphore()
            for nbr in (xpeer, ypeer):
                pl.semaphore_signal(
                    barrier_sem, inc=1, device_id=nbr,
                    device_id_type=pl.DeviceIdType.MESH,
                )
            pl.semaphore_wait(barrier_sem, 2)

        xsend[...] = x_ref[pl.ds(h0, H), :].astype(jnp.bfloat16)
        asend[...] = a_ref[pl.ds(h0, H), :]
        if _DO_P1:
            cx = pltpu.make_async_remote_copy(
                src_ref=xsend, dst_ref=xrecv,
                send_sem=send_sems.at[0], recv_sem=recv_sems.at[0],
                device_id=xpeer, device_id_type=pl.DeviceIdType.MESH,
            )
            cx.start()
            ca = pltpu.make_async_remote_copy(
                src_ref=asend, dst_ref=arecv,
                send_sem=send_sems.at[1], recv_sem=recv_sems.at[1],
                device_id=xpeer, device_id_type=pl.DeviceIdType.MESH,
            )
            ca.start()

        def moe_local_experts(xb, ab):
            acc = jnp.zeros((H, D), jnp.float32)
            for e in range(E):
                eid = my_x * E + e
                xm = jnp.where(ab == eid, xb, jnp.bfloat16(0.0))
                h = jnp.dot(xm, w1_ref[e].astype(jnp.bfloat16),
                            preferred_element_type=jnp.float32)
                h = jnp.maximum(h, 0.0).astype(jnp.bfloat16)
                acc = acc + jnp.dot(h, w2_ref[e].astype(jnp.bfloat16),
                                    preferred_element_type=jnp.float32)
            return acc

        if _DO_COMPUTE:
            p_own = moe_local_experts(xsend[...], asend[...])
        else:
            p_own = jnp.zeros((H, D), jnp.float32)

        if _DO_P1:
            cx.wait()
            ca.wait()
            xr, ar = xrecv[...], arecv[...]
        else:
            xr, ar = xsend[...], asend[...]

        if _DO_COMPUTE:
            prsend[...] = moe_local_experts(xr, ar).astype(jnp.bfloat16)
        else:
            prsend[...] = jnp.zeros((H, D), jnp.bfloat16)
        if _DO_P3:
            cp = pltpu.make_async_remote_copy(
                src_ref=prsend, dst_ref=prrecv,
                send_sem=send_sems.at[2], recv_sem=recv_sems.at[2],
                device_id=xpeer, device_id_type=pl.DeviceIdType.MESH,
            )
            cp.start()
            cp.wait()
            fin = p_own + prrecv[...].astype(jnp.float32)
        else:
            fin = p_own + prsend[...].astype(jnp.float32)

        fsend[...] = fin.astype(jnp.bfloat16)
        if _DO_P4:
            cf = pltpu.make_async_remote_copy(
                src_ref=fsend, dst_ref=frecv,
                send_sem=send_sems.at[3], recv_sem=recv_sems.at[3],
                device_id=ypeer, device_id_type=pl.DeviceIdType.MESH,
            )
            cf.start()
        out_ref[pl.ds(h0, H), :] = fin
        if _DO_P4:
            cf.wait()
            out_ref[pl.ds((1 - my_y) * H, H), :] = (
                frecv[...].astype(jnp.float32))
        else:
            out_ref[pl.ds((1 - my_y) * H, H), :] = (
                fsend[...].astype(jnp.float32))

    out_shape = jax.ShapeDtypeStruct((T, D), jnp.float32)
    return pl.pallas_call(
        body,
        out_shape=out_shape,
        in_specs=[
            pl.BlockSpec(memory_space=pltpu.VMEM),
            pl.BlockSpec(memory_space=pltpu.VMEM),
            pl.BlockSpec(memory_space=pltpu.VMEM),
            pl.BlockSpec(memory_space=pltpu.VMEM),
        ],
        out_specs=pl.BlockSpec(memory_space=pltpu.VMEM),
        scratch_shapes=[
            pltpu.VMEM((H, D), jnp.bfloat16),
            pltpu.VMEM((H, 1), jnp.int32),
            pltpu.VMEM((H, D), jnp.bfloat16),
            pltpu.VMEM((H, 1), jnp.int32),
            pltpu.VMEM((H, D), jnp.bfloat16),
            pltpu.VMEM((H, D), jnp.bfloat16),
            pltpu.VMEM((H, D), jnp.bfloat16),
            pltpu.VMEM((H, D), jnp.bfloat16),
            pltpu.SemaphoreType.DMA((4,)),
            pltpu.SemaphoreType.DMA((4,)),
        ],
        **({} if not _DO_BAR
           else dict(compiler_params=pltpu.CompilerParams(collective_id=0))),
    )(x, assign2, W1, W2)
